# baseline (speedup 1.0000x reference)
import functools

import jax
import jax.numpy as jnp
from jax import lax
from jax.experimental import pallas as pl
from jax.experimental.pallas import tpu as pltpu
from jax.experimental.pallas import tpu_sc as plsc

B = 16384
K = 128
N = 1024
UI = 1.0
IJ = 1.0

NC, NS, L = 2, 16, 16
NW = NC * NS
BPW = B // NW
CHUNK = 128
NCHUNK = BPW // CHUNK

HALF = N // 2
BLK = HALF * N


def _tables_body(gUI, gIU, gIJ, gJI, gIT, gTI, bU, bI, bT, alpha, p_blk):
    s = pl.program_id(0)
    m = s // 2
    h = s % 2
    dn = (((1,), (1,)), ((), ()))
    rows = pl.ds(h * HALF, HALF)

    def mk(lhs, rhs, bias, extra):
        def f():
            acc = lax.dot_general(lhs[rows, :], rhs[...], dn,
                                  preferred_element_type=jnp.float32)
            return acc + jnp.reshape(bias[rows], (HALF, 1)) + extra
        return f

    res = lax.switch(m, [
        mk(gUI, gIU, bU, alpha[0]),
        mk(gIJ, gJI, bI, 0.0),
        mk(gTI, gIT, bT, 0.0),
    ])
    p_blk[...] = jnp.reshape(res, (BLK,))


_tables = pl.pallas_call(
    _tables_body,
    grid=(6,),
    in_specs=[pl.BlockSpec((N, K), lambda s: (0, 0))] * 6
    + [pl.BlockSpec((N,), lambda s: (0,))] * 3
    + [pl.BlockSpec(memory_space=pltpu.SMEM)],
    out_specs=pl.BlockSpec((BLK,), lambda s: (s,)),
    out_shape=jax.ShapeDtypeStruct((3 * N * N,), jnp.float32),
)


_sc_mesh = plsc.VectorSubcoreMesh(core_axis_name="c", subcore_axis_name="s")


@functools.partial(
    pl.kernel,
    out_type=jax.ShapeDtypeStruct((NW, L), jnp.float32),
    mesh=_sc_mesh,
    scratch_types=[
        pltpu.VMEM((BPW,), jnp.int32),
        pltpu.VMEM((BPW,), jnp.int32),
        pltpu.VMEM((BPW,), jnp.int32),
        pltpu.VMEM((BPW,), jnp.int32),
        pltpu.VMEM((BPW,), jnp.float32),
        pltpu.VMEM((NCHUNK, CHUNK), jnp.int32),
        pltpu.VMEM((NCHUNK, CHUNK), jnp.int32),
        pltpu.VMEM((NCHUNK, CHUNK), jnp.int32),
        pltpu.VMEM((NCHUNK, CHUNK), jnp.float32),
        pltpu.VMEM((NCHUNK, CHUNK), jnp.float32),
        pltpu.VMEM((NCHUNK, CHUNK), jnp.float32),
        pltpu.VMEM((L,), jnp.float32),
        pltpu.SemaphoreType.DMA,
    ],
)
def _sc_loss(p_hbm, u_hbm, i_hbm, j_hbm, t_hbm, r_hbm,
             out_hbm, u_v, i_v, j_v, t_v, r_v, x1, x2, x3, g1, g2, g3,
             acc, sem):
    wid = lax.axis_index("s") * NC + lax.axis_index("c")
    base = wid * BPW
    pltpu.sync_copy(u_hbm.at[pl.ds(base, BPW)], u_v)
    pltpu.sync_copy(i_hbm.at[pl.ds(base, BPW)], i_v)
    pltpu.sync_copy(j_hbm.at[pl.ds(base, BPW)], j_v)
    pltpu.sync_copy(t_hbm.at[pl.ds(base, BPW)], t_v)
    pltpu.sync_copy(r_hbm.at[pl.ds(base, BPW)], r_v)

    for c in range(NCHUNK):
        for o in range(CHUNK // L):
            off = c * CHUNK + o * L
            sl = pl.ds(o * L, L)
            uu = u_v[pl.ds(off, L)]
            ii = i_v[pl.ds(off, L)]
            jj = j_v[pl.ds(off, L)]
            tt = t_v[pl.ds(off, L)]
            x1[c, sl] = uu * N + ii
            x2[c, sl] = ii * N + jj + (N * N)
            x3[c, sl] = tt * N + ii + (2 * N * N)

    copies = []
    for c in range(NCHUNK):
        copies.append(pltpu.async_copy(p_hbm.at[x1.at[c]], g1.at[c], sem))
        copies.append(pltpu.async_copy(p_hbm.at[x2.at[c]], g2.at[c], sem))
        copies.append(pltpu.async_copy(p_hbm.at[x3.at[c]], g3.at[c], sem))
    for cp in copies:
        cp.wait()

    total = jnp.zeros((L,), jnp.float32)
    for c in range(NCHUNK):
        for o in range(CHUNK // L):
            sl = pl.ds(o * L, L)
            pred = g1[c, sl] + g2[c, sl] + g3[c, sl]
            d = pred - r_v[pl.ds(c * CHUNK + o * L, L)]
            total = total + d * d
    acc[...] = total
    pltpu.sync_copy(acc, out_hbm.at[wid])


def kernel(sampleT, sampleU, sampleI, sampleJ, sampleR, alpha, betaU, betaI,
           betaT, gammaUI, gammaIU, gammaIJ, gammaJI, gammaIT, gammaTI):
    p = _tables(
        gammaUI, gammaIU, gammaIJ, gammaJI, gammaIT, gammaTI,
        betaU, betaI, betaT, alpha.reshape(1),
    )
    partials = _sc_loss(p, sampleU, sampleI, sampleJ, sampleT, sampleR)
    return 0.5 * jnp.sum(partials) / B


# speedup vs baseline: 1.1349x; 1.1349x over previous
import functools

import jax
import jax.numpy as jnp
from jax import lax
from jax.experimental import pallas as pl
from jax.experimental.pallas import tpu as pltpu
from jax.experimental.pallas import tpu_sc as plsc

B = 16384
K = 128
N = 1024
UI = 1.0
IJ = 1.0

NC, NS, L = 2, 16, 16
NW = NC * NS
BPW = B // NW
CHUNK = 128
NCHUNK = BPW // CHUNK

HALF = N // 2
BLK = HALF * N


def _tables_body(gUI, gIU, gIJ, gJI, gIT, gTI, bU, bI, bT, alpha, p):
    dn = (((1,), (1,)), ((), ()))
    a = alpha[0]
    p[pl.ds(0, N * N)] = jnp.reshape(
        lax.dot_general(gUI[...], gIU[...], dn,
                        preferred_element_type=jnp.float32)
        + UI * jnp.reshape(bU[...], (N, 1)) + a, (N * N,))
    p[pl.ds(N * N, N * N)] = jnp.reshape(
        IJ * lax.dot_general(gIJ[...], gJI[...], dn,
                             preferred_element_type=jnp.float32)
        + jnp.reshape(bI[...], (N, 1)), (N * N,))
    p[pl.ds(2 * N * N, N * N)] = jnp.reshape(
        lax.dot_general(gTI[...], gIT[...], dn,
                        preferred_element_type=jnp.float32)
        + jnp.reshape(bT[...], (N, 1)), (N * N,))


_tables = pl.pallas_call(
    _tables_body,
    in_specs=[pl.BlockSpec(memory_space=pltpu.VMEM)] * 9
    + [pl.BlockSpec(memory_space=pltpu.SMEM)],
    out_shape=jax.ShapeDtypeStruct((3 * N * N,), jnp.float32),
)


_sc_mesh = plsc.VectorSubcoreMesh(core_axis_name="c", subcore_axis_name="s")


@functools.partial(
    pl.kernel,
    out_type=jax.ShapeDtypeStruct((NW, L), jnp.float32),
    mesh=_sc_mesh,
    scratch_types=[
        pltpu.VMEM((BPW,), jnp.int32),
        pltpu.VMEM((BPW,), jnp.int32),
        pltpu.VMEM((BPW,), jnp.int32),
        pltpu.VMEM((BPW,), jnp.int32),
        pltpu.VMEM((BPW,), jnp.float32),
        pltpu.VMEM((NCHUNK, CHUNK), jnp.int32),
        pltpu.VMEM((NCHUNK, CHUNK), jnp.int32),
        pltpu.VMEM((NCHUNK, CHUNK), jnp.int32),
        pltpu.VMEM((NCHUNK, CHUNK), jnp.float32),
        pltpu.VMEM((NCHUNK, CHUNK), jnp.float32),
        pltpu.VMEM((NCHUNK, CHUNK), jnp.float32),
        pltpu.VMEM((L,), jnp.float32),
        pltpu.SemaphoreType.DMA,
    ],
)
def _sc_loss(p_hbm, u_hbm, i_hbm, j_hbm, t_hbm, r_hbm,
             out_hbm, u_v, i_v, j_v, t_v, r_v, x1, x2, x3, g1, g2, g3,
             acc, sem):
    wid = lax.axis_index("s") * NC + lax.axis_index("c")
    base = wid * BPW
    pltpu.sync_copy(u_hbm.at[pl.ds(base, BPW)], u_v)
    pltpu.sync_copy(i_hbm.at[pl.ds(base, BPW)], i_v)
    pltpu.sync_copy(j_hbm.at[pl.ds(base, BPW)], j_v)
    pltpu.sync_copy(t_hbm.at[pl.ds(base, BPW)], t_v)
    pltpu.sync_copy(r_hbm.at[pl.ds(base, BPW)], r_v)

    for c in range(NCHUNK):
        for o in range(CHUNK // L):
            off = c * CHUNK + o * L
            sl = pl.ds(o * L, L)
            uu = u_v[pl.ds(off, L)]
            ii = i_v[pl.ds(off, L)]
            jj = j_v[pl.ds(off, L)]
            tt = t_v[pl.ds(off, L)]
            x1[c, sl] = uu * N + ii
            x2[c, sl] = ii * N + jj + (N * N)
            x3[c, sl] = tt * N + ii + (2 * N * N)

    copies = []
    for c in range(NCHUNK):
        copies.append(pltpu.async_copy(p_hbm.at[x1.at[c]], g1.at[c], sem))
        copies.append(pltpu.async_copy(p_hbm.at[x2.at[c]], g2.at[c], sem))
        copies.append(pltpu.async_copy(p_hbm.at[x3.at[c]], g3.at[c], sem))
    for cp in copies:
        cp.wait()

    total = jnp.zeros((L,), jnp.float32)
    for c in range(NCHUNK):
        for o in range(CHUNK // L):
            sl = pl.ds(o * L, L)
            pred = g1[c, sl] + g2[c, sl] + g3[c, sl]
            d = pred - r_v[pl.ds(c * CHUNK + o * L, L)]
            total = total + d * d
    acc[...] = total
    pltpu.sync_copy(acc, out_hbm.at[wid])


def kernel(sampleT, sampleU, sampleI, sampleJ, sampleR, alpha, betaU, betaI,
           betaT, gammaUI, gammaIU, gammaIJ, gammaJI, gammaIT, gammaTI):
    p = _tables(
        gammaUI, gammaIU, gammaIJ, gammaJI, gammaIT, gammaTI,
        betaU, betaI, betaT, alpha.reshape(1),
    )
    partials = _sc_loss(p, sampleU, sampleI, sampleJ, sampleT, sampleR)
    return 0.5 * jnp.sum(partials) / B


# speedup vs baseline: 1.2708x; 1.1198x over previous
import functools

import jax
import jax.numpy as jnp
from jax import lax
from jax.experimental import pallas as pl
from jax.experimental.pallas import tpu as pltpu
from jax.experimental.pallas import tpu_sc as plsc

B = 16384
K = 128
N = 1024
UI = 1.0
IJ = 1.0

NC, NS, L = 2, 16, 16
NW = NC * NS
BPW = B // NW
CHUNK = 128
NCHUNK = BPW // CHUNK

HALF = N // 2
BLK = HALF * N


def _tables_body(gUI, gIU, gIJ, gJI, gIT, gTI, bU, bI, bT, alpha,
                 p1, p2, p3, s1, s2, s3, sem):
    dn = (((1,), (1,)), ((), ()))
    a = alpha[0]
    s1[...] = jnp.reshape(
        lax.dot_general(gUI[...], gIU[...], dn,
                        preferred_element_type=jnp.float32)
        + UI * jnp.reshape(bU[...], (N, 1)) + a, (N * N,))
    c1 = pltpu.make_async_copy(s1, p1, sem)
    c1.start()
    s2[...] = jnp.reshape(
        IJ * lax.dot_general(gIJ[...], gJI[...], dn,
                             preferred_element_type=jnp.float32)
        + jnp.reshape(bI[...], (N, 1)), (N * N,))
    c2 = pltpu.make_async_copy(s2, p2, sem)
    c2.start()
    s3[...] = jnp.reshape(
        lax.dot_general(gTI[...], gIT[...], dn,
                        preferred_element_type=jnp.float32)
        + jnp.reshape(bT[...], (N, 1)), (N * N,))
    c3 = pltpu.make_async_copy(s3, p3, sem)
    c3.start()
    c1.wait()
    c2.wait()
    c3.wait()


_tables = pl.pallas_call(
    _tables_body,
    in_specs=[pl.BlockSpec(memory_space=pltpu.VMEM)] * 9
    + [pl.BlockSpec(memory_space=pltpu.SMEM)],
    out_specs=[pl.BlockSpec(memory_space=pltpu.MemorySpace.HBM)] * 3,
    out_shape=[jax.ShapeDtypeStruct((N * N,), jnp.float32)] * 3,
    scratch_shapes=[pltpu.VMEM((N * N,), jnp.float32)] * 3
    + [pltpu.SemaphoreType.DMA],
)


_sc_mesh = plsc.VectorSubcoreMesh(core_axis_name="c", subcore_axis_name="s")


@functools.partial(
    pl.kernel,
    out_type=jax.ShapeDtypeStruct((NW, L), jnp.float32),
    mesh=_sc_mesh,
    scratch_types=[
        pltpu.VMEM((BPW,), jnp.int32),
        pltpu.VMEM((BPW,), jnp.int32),
        pltpu.VMEM((BPW,), jnp.int32),
        pltpu.VMEM((BPW,), jnp.int32),
        pltpu.VMEM((BPW,), jnp.float32),
        pltpu.VMEM((NCHUNK, CHUNK), jnp.int32),
        pltpu.VMEM((NCHUNK, CHUNK), jnp.int32),
        pltpu.VMEM((NCHUNK, CHUNK), jnp.int32),
        pltpu.VMEM((NCHUNK, CHUNK), jnp.float32),
        pltpu.VMEM((NCHUNK, CHUNK), jnp.float32),
        pltpu.VMEM((NCHUNK, CHUNK), jnp.float32),
        pltpu.VMEM((L,), jnp.float32),
        pltpu.SemaphoreType.DMA,
    ],
)
def _sc_loss(p1_hbm, p2_hbm, p3_hbm, u_hbm, i_hbm, j_hbm, t_hbm, r_hbm,
             out_hbm, u_v, i_v, j_v, t_v, r_v, x1, x2, x3, g1, g2, g3,
             acc, sem):
    wid = lax.axis_index("s") * NC + lax.axis_index("c")
    base = wid * BPW
    in_copies = [
        pltpu.async_copy(u_hbm.at[pl.ds(base, BPW)], u_v, sem),
        pltpu.async_copy(i_hbm.at[pl.ds(base, BPW)], i_v, sem),
        pltpu.async_copy(j_hbm.at[pl.ds(base, BPW)], j_v, sem),
        pltpu.async_copy(t_hbm.at[pl.ds(base, BPW)], t_v, sem),
        pltpu.async_copy(r_hbm.at[pl.ds(base, BPW)], r_v, sem),
    ]
    for cp in in_copies:
        cp.wait()

    copies = []
    for c in range(NCHUNK):
        for o in range(CHUNK // L):
            off = c * CHUNK + o * L
            sl = pl.ds(o * L, L)
            uu = u_v[pl.ds(off, L)]
            ii = i_v[pl.ds(off, L)]
            jj = j_v[pl.ds(off, L)]
            tt = t_v[pl.ds(off, L)]
            x1[c, sl] = uu * N + ii
            x2[c, sl] = ii * N + jj
            x3[c, sl] = tt * N + ii
        copies.append(pltpu.async_copy(p1_hbm.at[x1.at[c]], g1.at[c], sem))
        copies.append(pltpu.async_copy(p2_hbm.at[x2.at[c]], g2.at[c], sem))
        copies.append(pltpu.async_copy(p3_hbm.at[x3.at[c]], g3.at[c], sem))
    for cp in copies:
        cp.wait()

    total = jnp.zeros((L,), jnp.float32)
    for c in range(NCHUNK):
        for o in range(CHUNK // L):
            sl = pl.ds(o * L, L)
            pred = g1[c, sl] + g2[c, sl] + g3[c, sl]
            d = pred - r_v[pl.ds(c * CHUNK + o * L, L)]
            total = total + d * d
    acc[...] = total
    pltpu.sync_copy(acc, out_hbm.at[wid])


def kernel(sampleT, sampleU, sampleI, sampleJ, sampleR, alpha, betaU, betaI,
           betaT, gammaUI, gammaIU, gammaIJ, gammaJI, gammaIT, gammaTI):
    p1, p2, p3 = _tables(
        gammaUI, gammaIU, gammaIJ, gammaJI, gammaIT, gammaTI,
        betaU, betaI, betaT, alpha.reshape(1),
    )
    partials = _sc_loss(p1, p2, p3, sampleU, sampleI, sampleJ, sampleT,
                        sampleR)
    return 0.5 * jnp.sum(partials) / B
